# Initial kernel scaffold; baseline (speedup 1.0000x reference)
import jax
import jax.numpy as jnp
from jax.experimental import pallas as pl


def kernel(x, edge_index, W1, b1, W2, b2, W3, b3, Wh, bh):
    raise NotImplementedError("write your pallas kernel here")


# speedup vs baseline: 10.0279x; 10.0279x over previous
import functools

import jax
import jax.numpy as jnp
from jax import lax
from jax.experimental import pallas as pl
from jax.experimental.pallas import tpu as pltpu
from jax.experimental.pallas import tpu_sc as plsc

N, D, E = 10000, 128, 320000
NC, NS = 2, 16
NW = NC * NS
B = 128
CPT = -(-E // (NW * B))
EP = NW * B * CPT
EPT = EP // NW
NP = 10240
RPS = NP // NS
DEGW = 16

_mesh = plsc.VectorSubcoreMesh(
    core_axis_name="c", subcore_axis_name="s", num_cores=NC, num_subcores=NS)
_sc_params = pltpu.CompilerParams(use_tc_tiling_on_sc=False)


@functools.partial(
    pl.kernel,
    out_type=jax.ShapeDtypeStruct((NC, NP, DEGW), jnp.float32),
    mesh=_mesh,
    scratch_types=[
        pltpu.VMEM((B, DEGW), jnp.float32),
        pltpu.VMEM((CPT, B), jnp.int32),
        pltpu.VMEM_SHARED((NP, DEGW), jnp.float32),
    ],
    compiler_params=_sc_params,
)
def _deg_sc(dst_hbm, ones_hbm, zeros_hbm, out_hbm, ones_v, dst_v, acc):
    c = lax.axis_index("c")
    s = lax.axis_index("s")
    wid = c * NS + s
    pltpu.sync_copy(zeros_hbm.at[pl.ds(s * RPS, RPS)], acc.at[pl.ds(s * RPS, RPS)])
    pltpu.sync_copy(ones_hbm, ones_v)
    pltpu.sync_copy(dst_hbm.at[pl.ds(wid * CPT, CPT)], dst_v)
    plsc.subcore_barrier()

    def body(j, carry):
        pltpu.sync_copy(ones_v, acc.at[dst_v.at[j]], add=True)
        return carry

    lax.fori_loop(0, CPT, body, 0)
    plsc.subcore_barrier()
    pltpu.sync_copy(acc.at[pl.ds(s * RPS, RPS)],
                    out_hbm.at[c, pl.ds(s * RPS, RPS)])


@functools.partial(
    pl.kernel,
    out_type=jax.ShapeDtypeStruct((NC, NP, D), jnp.float32),
    mesh=_mesh,
    scratch_types=[
        pltpu.VMEM((CPT, B), jnp.int32),
        pltpu.VMEM((CPT, B), jnp.int32),
        pltpu.VMEM((B, D), jnp.float32),
        pltpu.VMEM_SHARED((NP, D), jnp.float32),
        pltpu.SemaphoreType.DMA,
    ],
    compiler_params=_sc_params,
)
def _agg_sc(g_hbm, src_hbm, dst_hbm, zeros_hbm, out_hbm,
            src_v, dst_v, buf, acc, sem):
    c = lax.axis_index("c")
    s = lax.axis_index("s")
    wid = c * NS + s
    pltpu.sync_copy(zeros_hbm.at[pl.ds(s * RPS, RPS)], acc.at[pl.ds(s * RPS, RPS)])
    pltpu.sync_copy(src_hbm.at[pl.ds(wid * CPT, CPT)], src_v)
    pltpu.sync_copy(dst_hbm.at[pl.ds(wid * CPT, CPT)], dst_v)
    plsc.subcore_barrier()

    def body(j, carry):
        pltpu.async_copy(g_hbm.at[src_v.at[j]], buf, sem).wait()
        pltpu.sync_copy(buf, acc.at[dst_v.at[j]], add=True)
        return carry

    lax.fori_loop(0, CPT, body, 0)
    plsc.subcore_barrier()
    pltpu.sync_copy(acc.at[pl.ds(s * RPS, RPS)],
                    out_hbm.at[c, pl.ds(s * RPS, RPS)])


NB = 10
RB = N // NB

_blk = pl.BlockSpec((RB, D), lambda i: (i, 0))
_blkdeg = pl.BlockSpec((RB, DEGW), lambda i: (i, 0))
_blkw = pl.BlockSpec((D, D), lambda i: (0, 0))
_blkb = pl.BlockSpec((1, D), lambda i: (0, 0))
_tc_params = pltpu.CompilerParams(dimension_semantics=("parallel",))


def _dinv_of(dp0_ref, dp1_ref):
    return lax.rsqrt(1.0 + dp0_ref[:, 0:1] + dp1_ref[:, 0:1])


def _tc_first_body(dp0_ref, dp1_ref, x_ref, w_ref, o_ref):
    dinv = _dinv_of(dp0_ref, dp1_ref)
    h = jnp.dot(x_ref[...], w_ref[...], preferred_element_type=jnp.float32)
    o_ref[...] = h * dinv


def _tc_mid_body(dp0_ref, dp1_ref, g_ref, s0_ref, s1_ref, w_ref, b_ref, o_ref):
    dinv = _dinv_of(dp0_ref, dp1_ref)
    z = dinv * (g_ref[...] + s0_ref[...] + s1_ref[...]) + b_ref[...]
    z = jnp.maximum(z, 0.0)
    h = jnp.dot(z, w_ref[...], preferred_element_type=jnp.float32)
    o_ref[...] = h * dinv


def _tc_last_body(dp0_ref, dp1_ref, g_ref, s0_ref, s1_ref, w_ref, b_ref,
                  bh_ref, o_ref):
    dinv = _dinv_of(dp0_ref, dp1_ref)
    z = dinv * (g_ref[...] + s0_ref[...] + s1_ref[...]) + b_ref[...]
    h = jnp.dot(z, w_ref[...], preferred_element_type=jnp.float32)
    o_ref[...] = h + bh_ref[...]


_out_nd = jax.ShapeDtypeStruct((N, D), jnp.float32)

_tc_first = pl.pallas_call(
    _tc_first_body,
    grid=(NB,),
    in_specs=[_blkdeg, _blkdeg, _blk, _blkw],
    out_specs=_blk,
    out_shape=_out_nd,
    compiler_params=_tc_params,
)

_tc_mid = pl.pallas_call(
    _tc_mid_body,
    grid=(NB,),
    in_specs=[_blkdeg, _blkdeg, _blk, _blk, _blk, _blkw, _blkb],
    out_specs=_blk,
    out_shape=_out_nd,
    compiler_params=_tc_params,
)

_tc_last = pl.pallas_call(
    _tc_last_body,
    grid=(NB,),
    in_specs=[_blkdeg, _blkdeg, _blk, _blk, _blk, _blkw, _blkb, _blkb],
    out_specs=_blk,
    out_shape=_out_nd,
    compiler_params=_tc_params,
)


def kernel(x, edge_index, W1, b1, W2, b2, W3, b3, Wh, bh):
    pad = EP - E
    src2 = jnp.concatenate(
        [edge_index[0], jnp.zeros((pad,), jnp.int32)]).reshape(EP // B, B)
    dst2 = jnp.concatenate(
        [edge_index[1], jnp.full((pad,), N, jnp.int32)]).reshape(EP // B, B)
    zeros_d = jnp.zeros((NP, D), jnp.float32)
    zeros_w = jnp.zeros((NP, DEGW), jnp.float32)
    ones_w = jnp.ones((B, DEGW), jnp.float32)
    b1r = b1.reshape(1, D)
    b2r = b2.reshape(1, D)
    b3r = b3.reshape(1, D)
    bhr = bh.reshape(1, D)

    degp = _deg_sc(dst2, ones_w, zeros_w)
    dp0, dp1 = degp[0, :N], degp[1, :N]

    g1 = _tc_first(dp0, dp1, x, W1)
    s1 = _agg_sc(g1, src2, dst2, zeros_d)
    g2 = _tc_mid(dp0, dp1, g1, s1[0, :N], s1[1, :N], W2, b1r)
    s2 = _agg_sc(g2, src2, dst2, zeros_d)
    g3 = _tc_mid(dp0, dp1, g2, s2[0, :N], s2[1, :N], W3, b2r)
    s3 = _agg_sc(g3, src2, dst2, zeros_d)
    out = _tc_last(dp0, dp1, g3, s3[0, :N], s3[1, :N], Wh, b3r, bhr)
    return out
